# baseline (speedup 1.0000x reference)
import functools

import jax
import jax.numpy as jnp
from jax import lax
from jax.experimental import pallas as pl
from jax.experimental.pallas import tpu as pltpu
from jax.experimental.pallas import tpu_sc as plsc

C = 3
ID_VOCAB = 100000
D = 128
OUT_LEN = 200
L = 150
T = 50
NSL = D // 16
HB = 8
IDS_ROWS = 2 * HB


def kernel(input_ids, input_vecs, mask, id_embed, type_embed, pos_embed):
    del mask
    B = input_ids.shape[0]
    info = plsc.get_sparse_core_info()
    NC, NS = info.num_cores, info.num_subcores
    NW = NC * NS
    rows_per_w = B // NW
    n_hb = rows_per_w // HB

    ids_flat = input_ids.reshape(B * L)
    type_pad = jnp.zeros((8, D), jnp.float32).at[:2].set(type_embed)

    mesh = plsc.VectorSubcoreMesh(core_axis_name="c", subcore_axis_name="s")

    @functools.partial(
        pl.kernel,
        mesh=mesh,
        out_type=jax.ShapeDtypeStruct((B * OUT_LEN, D), jnp.float32),
        scratch_types=[
            pltpu.VMEM((IDS_ROWS * L,), jnp.int32),
            pltpu.VMEM((80,), jnp.int32),
            pltpu.VMEM((80,), jnp.int32),
            pltpu.VMEM((70,), jnp.int32),
            pltpu.VMEM((70,), jnp.int32),
            pltpu.VMEM((OUT_LEN, D), jnp.float32),
            pltpu.VMEM((OUT_LEN, D), jnp.float32),
            pltpu.VMEM((OUT_LEN, D), jnp.float32),
            pltpu.VMEM((OUT_LEN, D), jnp.float32),
            pltpu.VMEM((OUT_LEN, D), jnp.float32),
            pltpu.SemaphoreType.DMA,
            pltpu.SemaphoreType.DMA,
            pltpu.SemaphoreType.DMA,
            pltpu.SemaphoreType.DMA,
            pltpu.SemaphoreType.DMA,
        ],
    )
    def sc_kernel(ids_hbm, vecs_hbm, table_hbm, type_hbm, pos_hbm,
                  out_hbm, idsr, gidx_a0, gidx_a1, gidx_b0, gidx_b1,
                  stag0, stag1, outv0, outv1, bias_v,
                  gsem0, gsem1, wsem0, wsem1, isem):
        gidx_a = (gidx_a0, gidx_a1)
        gidx_b = (gidx_b0, gidx_b1)
        staging = (stag0, stag1)
        out_v = (outv0, outv1)
        gsem = (gsem0, gsem1)
        wsem = (wsem0, wsem1)

        wid = lax.axis_index("s") * NC + lax.axis_index("c")
        row_base = wid * rows_per_w

        for c0, cn in ((0, 48), (48, 48), (96, 48), (144, 56)):
            pltpu.sync_copy(pos_hbm.at[pl.ds(c0, cn)],
                            outv0.at[pl.ds(c0, cn)])
        pltpu.sync_copy(type_hbm, stag1.at[pl.ds(0, 8)])

        def bias_body(i, carry):
            for j in range(4):
                t = 1 if j == 3 else 0
                p = i * 4 + j
                for s in range(NSL):
                    sl = pl.ds(s * 16, 16)
                    bias_v[p, sl] = outv0[p, sl] + stag1[t, sl]
            return carry

        lax.fori_loop(0, T, bias_body, 0)

        A_STARTS = [0, 16, 32, 48, 64]
        B_STARTS = [80, 96, 112, 128, 134]

        def fire_gathers(q, b, lrow):
            for starts, idx_ref in ((A_STARTS, gidx_a[q]),
                                    (B_STARTS, gidx_b[q])):
                for s0 in starts:
                    v = idsr[pl.ds(lrow * L + s0, 16)]
                    l16 = lax.iota(jnp.int32, 16) + s0
                    o = lax.rem(l16, C) * ID_VOCAB
                    e = jnp.where(v != 0, v + o, v)
                    idx_ref[pl.ds(s0 - starts[0], 16)] = e
            h1 = pltpu.async_copy(table_hbm.at[gidx_a[q]],
                                  staging[q].at[pl.ds(0, 80)], gsem[q])
            h2 = pltpu.async_copy(table_hbm.at[gidx_b[q]],
                                  staging[q].at[pl.ds(80, 70)], gsem[q])
            h3 = pltpu.async_copy(vecs_hbm.at[b],
                                  staging[q].at[pl.ds(L, T)], gsem[q])
            return (h1, h2, h3)

        def drain_gathers(q):
            pltpu.make_async_copy(table_hbm.at[gidx_a[q]],
                                  staging[q].at[pl.ds(0, 80)], gsem[q]).wait()
            pltpu.make_async_copy(table_hbm.at[gidx_b[q]],
                                  staging[q].at[pl.ds(80, 70)], gsem[q]).wait()
            pltpu.make_async_copy(vecs_hbm.at[0],
                                  staging[q].at[pl.ds(L, T)], gsem[q]).wait()

        pltpu.sync_copy(ids_hbm.at[pl.ds(row_base * L, HB * L)],
                        idsr.at[pl.ds(0, HB * L)])
        for r01 in range(2):
            fire_gathers(r01, row_base + r01, r01)

        def hb_body(hb, carry):
            hb0 = hb * HB

            @pl.when(hb > 0)
            def _():
                for q in range(2):
                    pltpu.make_async_copy(
                        out_v[q].at[pl.ds(0, 96)],
                        out_hbm.at[pl.ds(0, 96)], wsem[q]).wait()
                    pltpu.make_async_copy(
                        out_v[q].at[pl.ds(96, 104)],
                        out_hbm.at[pl.ds(96, 104)], wsem[q]).wait()

            ids_half = lax.rem(hb + 1, 2) * HB

            @pl.when(hb < n_hb - 1)
            def _():
                src = ids_hbm.at[pl.ds((row_base + hb0 + HB) * L,
                                       HB * L)]
                for h in range(2):
                    @pl.when(ids_half == h * HB)
                    def _():
                        pltpu.async_copy(
                            src, idsr.at[pl.ds(h * HB * L, HB * L)], isem)

            whandles = [None, None]
            for r in range(HB):
                q = r % 2
                gr = hb0 + r
                b = row_base + gr
                if r >= 2:
                    for wh in whandles[q]:
                        wh.wait()
                drain_gathers(q)

                @plsc.parallel_loop(0, T, unroll=2)
                def item_body(i):
                    for j in range(3):
                        lj = i * 3 + j
                        pj = i * 4 + j
                        for s in range(NSL):
                            sl = pl.ds(s * 16, 16)
                            out_v[q][pj, sl] = (staging[q][lj, sl]
                                                + bias_v[pj, sl])
                    pv = i * 4 + 3
                    for s in range(NSL):
                        sl = pl.ds(s * 16, 16)
                        out_v[q][pv, sl] = (staging[q][L + i, sl]
                                            + bias_v[pv, sl])

                wh1 = pltpu.async_copy(
                    out_v[q].at[pl.ds(0, 96)],
                    out_hbm.at[pl.ds(b * OUT_LEN, 96)], wsem[q])
                wh2 = pltpu.async_copy(
                    out_v[q].at[pl.ds(96, 104)],
                    out_hbm.at[pl.ds(b * OUT_LEN + 96, 104)], wsem[q])
                whandles[q] = (wh1, wh2)

                if r == 6:
                    @pl.when(hb < n_hb - 1)
                    def _():
                        pltpu.make_async_copy(
                            ids_hbm.at[pl.ds(0, HB * L)],
                            idsr.at[pl.ds(0, HB * L)], isem).wait()

                if r < HB - 2:
                    fire_gathers(q, b + 2, lax.rem(gr + 2, IDS_ROWS))
                else:
                    @pl.when(hb < n_hb - 1)
                    def _():
                        fire_gathers(q, b + 2, lax.rem(gr + 2, IDS_ROWS))
            return carry

        lax.fori_loop(0, n_hb, hb_body, 0)
        for q in range(2):
            pltpu.make_async_copy(out_v[q].at[pl.ds(0, 96)],
                                  out_hbm.at[pl.ds(0, 96)], wsem[q]).wait()
            pltpu.make_async_copy(out_v[q].at[pl.ds(96, 104)],
                                  out_hbm.at[pl.ds(96, 104)], wsem[q]).wait()

    out = sc_kernel(ids_flat, input_vecs, id_embed, type_pad, pos_embed)
    return out.reshape(B, OUT_LEN, D)


# speedup vs baseline: 1.0371x; 1.0371x over previous
import functools

import jax
import jax.numpy as jnp
from jax import lax
from jax.experimental import pallas as pl
from jax.experimental.pallas import tpu as pltpu
from jax.experimental.pallas import tpu_sc as plsc

C = 3
ID_VOCAB = 100000
D = 128
OUT_LEN = 200
L = 150
T = 50
NSL = D // 16
HB = 8
IDS_ROWS = 2 * HB


def kernel(input_ids, input_vecs, mask, id_embed, type_embed, pos_embed):
    del mask
    B = input_ids.shape[0]
    info = plsc.get_sparse_core_info()
    NC, NS = info.num_cores, info.num_subcores
    NW = NC * NS
    rows_per_w = B // NW
    n_hb = rows_per_w // HB

    ids_flat = input_ids.reshape(B * L)
    type_pad = jnp.zeros((8, D), jnp.float32).at[:2].set(type_embed)

    mesh = plsc.VectorSubcoreMesh(core_axis_name="c", subcore_axis_name="s")

    @functools.partial(
        pl.kernel,
        mesh=mesh,
        out_type=jax.ShapeDtypeStruct((B * OUT_LEN, D), jnp.float32),
        scratch_types=[
            pltpu.VMEM((IDS_ROWS * L,), jnp.int32),
            pltpu.VMEM((80,), jnp.int32),
            pltpu.VMEM((80,), jnp.int32),
            pltpu.VMEM((70,), jnp.int32),
            pltpu.VMEM((70,), jnp.int32),
            pltpu.VMEM((OUT_LEN, D), jnp.float32),
            pltpu.VMEM((OUT_LEN, D), jnp.float32),
            pltpu.VMEM((OUT_LEN, D), jnp.float32),
            pltpu.VMEM((OUT_LEN, D), jnp.float32),
            pltpu.VMEM((OUT_LEN, D), jnp.float32),
            pltpu.SemaphoreType.DMA,
            pltpu.SemaphoreType.DMA,
            pltpu.SemaphoreType.DMA,
            pltpu.SemaphoreType.DMA,
            pltpu.SemaphoreType.DMA,
        ],
    )
    def sc_kernel(ids_hbm, vecs_hbm, table_hbm, type_hbm, pos_hbm,
                  out_hbm, idsr, gidx_a0, gidx_a1, gidx_b0, gidx_b1,
                  stag0, stag1, outv0, outv1, bias_v,
                  gsem0, gsem1, wsem0, wsem1, isem):
        gidx_a = (gidx_a0, gidx_a1)
        gidx_b = (gidx_b0, gidx_b1)
        staging = (stag0, stag1)
        out_v = (outv0, outv1)
        gsem = (gsem0, gsem1)
        wsem = (wsem0, wsem1)

        wid = lax.axis_index("s") * NC + lax.axis_index("c")
        row_base = wid * rows_per_w

        for c0, cn in ((0, 48), (48, 48), (96, 48), (144, 56)):
            pltpu.sync_copy(pos_hbm.at[pl.ds(c0, cn)],
                            outv0.at[pl.ds(c0, cn)])
        pltpu.sync_copy(type_hbm, stag1.at[pl.ds(0, 8)])

        def bias_body(i, carry):
            for j in range(4):
                t = 1 if j == 3 else 0
                p = i * 4 + j
                for s in range(NSL):
                    sl = pl.ds(s * 16, 16)
                    bias_v[p, sl] = outv0[p, sl] + stag1[t, sl]
            return carry

        lax.fori_loop(0, T, bias_body, 0)

        A_STARTS = [0, 16, 32, 48, 64]
        B_STARTS = [80, 96, 112, 128, 134]

        def fire_gathers(q, b, lrow):
            for starts, idx_ref in ((A_STARTS, gidx_a[q]),
                                    (B_STARTS, gidx_b[q])):
                for s0 in starts:
                    v = idsr[pl.ds(lrow * L + s0, 16)]
                    l16 = lax.iota(jnp.int32, 16) + s0
                    o = lax.rem(l16, C) * ID_VOCAB
                    e = jnp.where(v != 0, v + o, v)
                    idx_ref[pl.ds(s0 - starts[0], 16)] = e
            h1 = pltpu.async_copy(table_hbm.at[gidx_a[q]],
                                  staging[q].at[pl.ds(0, 80)], gsem[q])
            h2 = pltpu.async_copy(table_hbm.at[gidx_b[q]],
                                  staging[q].at[pl.ds(80, 70)], gsem[q])
            h3 = pltpu.async_copy(vecs_hbm.at[b],
                                  staging[q].at[pl.ds(L, T)], gsem[q])
            return (h1, h2, h3)

        def drain_gathers(q):
            pltpu.make_async_copy(table_hbm.at[gidx_a[q]],
                                  staging[q].at[pl.ds(0, 80)], gsem[q]).wait()
            pltpu.make_async_copy(table_hbm.at[gidx_b[q]],
                                  staging[q].at[pl.ds(80, 70)], gsem[q]).wait()
            pltpu.make_async_copy(vecs_hbm.at[0],
                                  staging[q].at[pl.ds(L, T)], gsem[q]).wait()

        pltpu.sync_copy(ids_hbm.at[pl.ds(row_base * L, HB * L)],
                        idsr.at[pl.ds(0, HB * L)])
        for r01 in range(2):
            fire_gathers(r01, row_base + r01, r01)

        def hb_body(hb, carry):
            hb0 = hb * HB

            @pl.when(hb > 0)
            def _():
                for q in range(2):
                    pltpu.make_async_copy(
                        out_v[q], out_hbm.at[pl.ds(0, OUT_LEN)],
                        wsem[q]).wait()

            ids_half = lax.rem(hb + 1, 2) * HB

            @pl.when(hb < n_hb - 1)
            def _():
                src = ids_hbm.at[pl.ds((row_base + hb0 + HB) * L,
                                       HB * L)]
                for h in range(2):
                    @pl.when(ids_half == h * HB)
                    def _():
                        pltpu.async_copy(
                            src, idsr.at[pl.ds(h * HB * L, HB * L)], isem)

            whandles = [None, None]
            for r in range(HB):
                q = r % 2
                gr = hb0 + r
                b = row_base + gr
                if r >= 2:
                    for wh in whandles[q]:
                        wh.wait()
                drain_gathers(q)

                @plsc.parallel_loop(0, T, unroll=1)
                def item_body(i):
                    for j in range(3):
                        lj = i * 3 + j
                        pj = i * 4 + j
                        for s in range(NSL):
                            sl = pl.ds(s * 16, 16)
                            out_v[q][pj, sl] = (staging[q][lj, sl]
                                                + bias_v[pj, sl])
                    pv = i * 4 + 3
                    for s in range(NSL):
                        sl = pl.ds(s * 16, 16)
                        out_v[q][pv, sl] = (staging[q][L + i, sl]
                                            + bias_v[pv, sl])

                wh = pltpu.async_copy(
                    out_v[q], out_hbm.at[pl.ds(b * OUT_LEN, OUT_LEN)],
                    wsem[q])
                whandles[q] = (wh,)

                if r == 6:
                    @pl.when(hb < n_hb - 1)
                    def _():
                        pltpu.make_async_copy(
                            ids_hbm.at[pl.ds(0, HB * L)],
                            idsr.at[pl.ds(0, HB * L)], isem).wait()

                if r < HB - 2:
                    fire_gathers(q, b + 2, lax.rem(gr + 2, IDS_ROWS))
                else:
                    @pl.when(hb < n_hb - 1)
                    def _():
                        fire_gathers(q, b + 2, lax.rem(gr + 2, IDS_ROWS))
            return carry

        lax.fori_loop(0, n_hb, hb_body, 0)
        for q in range(2):
            pltpu.make_async_copy(out_v[q], out_hbm.at[pl.ds(0, OUT_LEN)],
                                  wsem[q]).wait()

    out = sc_kernel(ids_flat, input_vecs, id_embed, type_pad, pos_embed)
    return out.reshape(B, OUT_LEN, D)
